# Initial kernel scaffold; baseline (speedup 1.0000x reference)
import jax
import jax.numpy as jnp
from jax.experimental import pallas as pl


def kernel(x, W_enc, b_enc, codebook):
    raise NotImplementedError("write your pallas kernel here")


# speedup vs baseline: 2.2572x; 2.2572x over previous
import jax
import jax.numpy as jnp
from jax.experimental import pallas as pl
from jax.experimental.pallas import tpu as pltpu

_P = 16
_HP = 32
_NTOK = _HP * _HP
_K = 256
_D = 32
_C = 3


def _patchify_channel(xc, masks):
    v = xc
    for m, (recv_dn, recv_up) in masks.items():
        r_dn = pltpu.roll(pltpu.roll(v, m, 0), 512 - 16 * m, 1)
        r_up = pltpu.roll(pltpu.roll(v, 512 - m, 0), 16 * m, 1)
        v = jnp.where(recv_dn, r_dn, jnp.where(recv_up, r_up, v))
    t = (v.reshape(_HP, 2, 8, 4, 128)
          .transpose(0, 3, 2, 1, 4)
          .reshape(_NTOK, _P * _P))
    return t


def _vq_encode_kernel(x_ref, w_ref, cbm2_ref, cb2_ref, out_ref):
    si = jax.lax.broadcasted_iota(jnp.int32, (512, 512), 0) % 8
    gi = (jax.lax.broadcasted_iota(jnp.int32, (512, 512), 1) // 16) % 8
    masks = {}
    for m in (4, 2, 1):
        sq = (si // m) % 2
        gq = (gi // m) % 2
        masks[m] = ((sq == 1) & (gq == 0), (sq == 0) & (gq == 1))
    z = jnp.zeros((_NTOK, _D), jnp.float32)
    for c in range(_C):
        t = _patchify_channel(x_ref[0, c], masks)
        z = z + jax.lax.dot_general(
            t, w_ref[c], (((1,), (0,)), ((), ())),
            preferred_element_type=jnp.float32,
            precision=jax.lax.Precision.DEFAULT)
    d = jax.lax.dot_general(
        z, cbm2_ref[...], (((1,), (1,)), ((), ())),
        preferred_element_type=jnp.float32,
        precision=jax.lax.Precision.DEFAULT) + cb2_ref[...]
    out_ref[0, 0] = jnp.argmin(d, axis=-1).astype(jnp.int32)


def kernel(x, W_enc, b_enc, codebook):
    B, C, H, W = x.shape
    w3 = W_enc.reshape(_C, _P * _P, _D)
    cbm2 = -2.0 * codebook
    cb2 = (jnp.sum(codebook * codebook, axis=-1)
           - 2.0 * (codebook @ b_enc)).reshape(1, _K)
    out = pl.pallas_call(
        _vq_encode_kernel,
        grid=(B,),
        in_specs=[
            pl.BlockSpec((1, _C, H, W), lambda b: (b, 0, 0, 0)),
            pl.BlockSpec((_C, _P * _P, _D), lambda b: (0, 0, 0)),
            pl.BlockSpec((_K, _D), lambda b: (0, 0)),
            pl.BlockSpec((1, _K), lambda b: (0, 0)),
        ],
        out_specs=pl.BlockSpec((1, 1, _NTOK), lambda b: (b, 0, 0)),
        out_shape=jax.ShapeDtypeStruct((B, 1, _NTOK), jnp.int32),
    )(x, w3, cbm2, cb2)
    return out.reshape(B, _NTOK).astype(jnp.uint8)


# speedup vs baseline: 3.0823x; 1.3656x over previous
import jax
import jax.numpy as jnp
from jax.experimental import pallas as pl
from jax.experimental.pallas import tpu as pltpu

_P = 16
_HP = 32
_NTOK = _HP * _HP
_K = 256
_D = 32
_C = 3


def _patchify_channel(xc, masks):
    x3 = xc.reshape(64, 8, 512)
    panels = []
    for p in range(4):
        u = x3[:, :, 128 * p:128 * (p + 1)]
        for m, (recv_dn, recv_up) in masks.items():
            r_dn = pltpu.roll(pltpu.roll(u, m, 1), 128 - 16 * m, 2)
            r_up = pltpu.roll(pltpu.roll(u, 8 - m, 1), 16 * m, 2)
            u = jnp.where(recv_dn[None], r_dn,
                          jnp.where(recv_up[None], r_up, u))
        panels.append(u)
    v = jnp.concatenate(panels, axis=2).reshape(512, 512)
    t = (v.reshape(_HP, 2, 8, 4, 128)
          .transpose(0, 3, 2, 1, 4)
          .reshape(_NTOK, _P * _P))
    return t


def _vq_encode_kernel(x_ref, w_ref, cbm2_ref, cb2_ref, out_ref):
    si = jax.lax.broadcasted_iota(jnp.int32, (8, 128), 0)
    gi = jax.lax.broadcasted_iota(jnp.int32, (8, 128), 1) // 16
    masks = {}
    for m in (4, 2, 1):
        sq = (si // m) % 2
        gq = (gi // m) % 2
        masks[m] = ((sq == 1) & (gq == 0), (sq == 0) & (gq == 1))
    z = jnp.zeros((_NTOK, _D), jnp.float32)
    for c in range(_C):
        t = _patchify_channel(x_ref[0, c], masks)
        z = z + jax.lax.dot_general(
            t, w_ref[c], (((1,), (0,)), ((), ())),
            preferred_element_type=jnp.float32,
            precision=jax.lax.Precision.DEFAULT)
    d = jax.lax.dot_general(
        z, cbm2_ref[...], (((1,), (1,)), ((), ())),
        preferred_element_type=jnp.float32,
        precision=jax.lax.Precision.DEFAULT) + cb2_ref[...]
    out_ref[0, 0] = jnp.argmin(d, axis=-1).astype(jnp.int32)


def kernel(x, W_enc, b_enc, codebook):
    B, C, H, W = x.shape
    w3 = W_enc.reshape(_C, _P * _P, _D)
    cbm2 = -2.0 * codebook
    cb2 = (jnp.sum(codebook * codebook, axis=-1)
           - 2.0 * (codebook @ b_enc)).reshape(1, _K)
    out = pl.pallas_call(
        _vq_encode_kernel,
        grid=(B,),
        in_specs=[
            pl.BlockSpec((1, _C, H, W), lambda b: (b, 0, 0, 0)),
            pl.BlockSpec((_C, _P * _P, _D), lambda b: (0, 0, 0)),
            pl.BlockSpec((_K, _D), lambda b: (0, 0)),
            pl.BlockSpec((1, _K), lambda b: (0, 0)),
        ],
        out_specs=pl.BlockSpec((1, 1, _NTOK), lambda b: (b, 0, 0)),
        out_shape=jax.ShapeDtypeStruct((B, 1, _NTOK), jnp.int32),
    )(x, w3, cbm2, cb2)
    return out.reshape(B, _NTOK).astype(jnp.uint8)


# speedup vs baseline: 4.0995x; 1.3300x over previous
import jax
import jax.numpy as jnp
from jax.experimental import pallas as pl
from jax.experimental.pallas import tpu as pltpu

_P = 16
_HP = 32
_NTOK = _HP * _HP
_K = 256
_D = 32
_C = 3


def _patchify_channel(xc, lane_masks):
    x3 = xc.reshape(64, 8, 512)
    panels = []
    for p in range(4):
        u = x3[:, :, 128 * p:128 * (p + 1)]
        u = pltpu.roll(u, 0, 2, stride=16, stride_axis=1)
        for b, mb in lane_masks:
            u = jnp.where(mb[None], pltpu.roll(u, 8 - b, 1), u)
        u = pltpu.roll(u, 8 - 1, 1)
        u = pltpu.roll(u, 16, 2)
        u = pltpu.roll(u, 0, 2, stride=16, stride_axis=1)
        panels.append(u)
    v = jnp.concatenate(panels, axis=2)
    t = (v.reshape(_HP, 2, 8, 4, 128)
          .transpose(0, 3, 2, 1, 4)
          .reshape(_NTOK, _P * _P))
    return t


def _vq_encode_kernel(x_ref, w_ref, cbm2_ref, cb2_ref, out_ref):
    gi = jax.lax.broadcasted_iota(jnp.int32, (8, 128), 1) // 16
    lane_masks = [(b, (gi // b) % 2 == 1) for b in (4, 2, 1)]
    z = jnp.zeros((_NTOK, _D), jnp.float32)
    for c in range(_C):
        t = _patchify_channel(x_ref[0, c], lane_masks)
        z = z + jax.lax.dot_general(
            t, w_ref[c], (((1,), (0,)), ((), ())),
            preferred_element_type=jnp.float32,
            precision=jax.lax.Precision.DEFAULT)
    d = jax.lax.dot_general(
        z, cbm2_ref[...], (((1,), (1,)), ((), ())),
        preferred_element_type=jnp.float32,
        precision=jax.lax.Precision.DEFAULT) + cb2_ref[...]
    out_ref[0, 0] = jnp.argmin(d, axis=-1).astype(jnp.int32)


def kernel(x, W_enc, b_enc, codebook):
    B, C, H, W = x.shape
    w3 = W_enc.reshape(_C, _P * _P, _D)
    cbm2 = -2.0 * codebook
    cb2 = (jnp.sum(codebook * codebook, axis=-1)
           - 2.0 * (codebook @ b_enc)).reshape(1, _K)
    out = pl.pallas_call(
        _vq_encode_kernel,
        grid=(B,),
        in_specs=[
            pl.BlockSpec((1, _C, H, W), lambda b: (b, 0, 0, 0)),
            pl.BlockSpec((_C, _P * _P, _D), lambda b: (0, 0, 0)),
            pl.BlockSpec((_K, _D), lambda b: (0, 0)),
            pl.BlockSpec((1, _K), lambda b: (0, 0)),
        ],
        out_specs=pl.BlockSpec((1, 1, _NTOK), lambda b: (b, 0, 0)),
        out_shape=jax.ShapeDtypeStruct((B, 1, _NTOK), jnp.int32),
    )(x, w3, cbm2, cb2)
    return out.reshape(B, _NTOK // 8, 8)[:, :, ::-1].reshape(B, _NTOK).astype(jnp.uint8)


# speedup vs baseline: 4.8026x; 1.1715x over previous
import jax
import jax.numpy as jnp
from jax.experimental import pallas as pl
from jax.experimental.pallas import tpu as pltpu

_P = 16
_HP = 32
_NTOK = _HP * _HP
_K = 256
_D = 32
_C = 3


def _patchify_channel(xc, lane_masks):
    x3 = xc.reshape(64, 8, 512)
    panels = []
    for p in range(4):
        u = x3[:, :, 128 * p:128 * (p + 1)]
        u = pltpu.roll(u, 0, 2, stride=16, stride_axis=1)
        for b, mb in lane_masks:
            u = jnp.where(mb[None], pltpu.roll(u, 8 - b, 1), u)
        u = pltpu.roll(u, 8 - 1, 1)
        u = pltpu.roll(u, 16, 2)
        u = pltpu.roll(u, 0, 2, stride=16, stride_axis=1)
        panels.append(u)
    v = jnp.concatenate(panels, axis=2)
    t = jnp.concatenate(
        [v.reshape(_HP, 2, 8, 512)[:, k].reshape(_HP, 8, 4, 128)
              .transpose(0, 2, 1, 3)
              .reshape(_NTOK, 128) for k in (0, 1)], axis=1)
    return t


def _vq_encode_kernel(x_ref, w_ref, cbm2_ref, cb2_ref, out_ref):
    gi = jax.lax.broadcasted_iota(jnp.int32, (8, 128), 1) // 16
    lane_masks = [(b, (gi // b) % 2 == 1) for b in (4, 2, 1)]
    z = jnp.zeros((_NTOK, _D), jnp.float32)
    for c in range(_C):
        t = _patchify_channel(x_ref[0, c], lane_masks)
        z = z + jax.lax.dot_general(
            t, w_ref[c], (((1,), (0,)), ((), ())),
            preferred_element_type=jnp.float32,
            precision=jax.lax.Precision.DEFAULT)
    d = jax.lax.dot_general(
        z, cbm2_ref[...], (((1,), (1,)), ((), ())),
        preferred_element_type=jnp.float32,
        precision=jax.lax.Precision.DEFAULT) + cb2_ref[...]
    out_ref[0, 0] = jnp.argmin(d, axis=-1).astype(jnp.int32)


def kernel(x, W_enc, b_enc, codebook):
    B, C, H, W = x.shape
    w3 = W_enc.reshape(_C, _P * _P, _D)
    cbm2 = -2.0 * codebook
    cb2 = (jnp.sum(codebook * codebook, axis=-1)
           - 2.0 * (codebook @ b_enc)).reshape(1, _K)
    out = pl.pallas_call(
        _vq_encode_kernel,
        grid=(B,),
        in_specs=[
            pl.BlockSpec((1, _C, H, W), lambda b: (b, 0, 0, 0)),
            pl.BlockSpec((_C, _P * _P, _D), lambda b: (0, 0, 0)),
            pl.BlockSpec((_K, _D), lambda b: (0, 0)),
            pl.BlockSpec((1, _K), lambda b: (0, 0)),
        ],
        out_specs=pl.BlockSpec((1, 1, _NTOK), lambda b: (b, 0, 0)),
        out_shape=jax.ShapeDtypeStruct((B, 1, _NTOK), jnp.int32),
    )(x, w3, cbm2, cb2)
    return out.reshape(B, _NTOK // 8, 8)[:, :, ::-1].reshape(B, _NTOK).astype(jnp.uint8)


# speedup vs baseline: 6.0314x; 1.2559x over previous
import jax
import jax.numpy as jnp
from jax.experimental import pallas as pl
from jax.experimental.pallas import tpu as pltpu

_P = 16
_HP = 32
_NTOK = _HP * _HP
_K = 256
_D = 32
_C = 3


def _patchify_channel(xc, lane_masks):
    x3 = xc.reshape(64, 8, 512)
    panels = []
    for p in range(4):
        u = x3[:, :, 128 * p:128 * (p + 1)]
        u = pltpu.roll(u, 0, 2, stride=16, stride_axis=1)
        for b, mb in lane_masks:
            u = jnp.where(mb[None], pltpu.roll(u, 8 - b, 1), u)
        u = pltpu.roll(u, 8 - 1, 1)
        u = pltpu.roll(u, 16, 2)
        u = pltpu.roll(u, 0, 2, stride=16, stride_axis=1)
        panels.append(u)
    v = jnp.concatenate(panels, axis=2)
    t = jnp.concatenate(
        [v.reshape(_HP, 2, 8, 512)[:, k].reshape(_HP, 8, 4, 128)
              .transpose(0, 2, 1, 3)
              .reshape(_NTOK, 128) for k in (0, 1)], axis=1)
    return t


def _vq_encode_kernel(x_ref, w_ref, cbm2_ref, cb2_ref, out_ref):
    gi = jax.lax.broadcasted_iota(jnp.int32, (8, 128), 1) // 16
    lane_masks = [(b, (gi // b) % 2 == 1) for b in (4, 2, 1)]
    z = jnp.zeros((_NTOK, _D), jnp.float32)
    for c in range(_C):
        t = _patchify_channel(x_ref[0, c], lane_masks)
        z = z + jax.lax.dot_general(
            t, w_ref[c], (((1,), (0,)), ((), ())),
            preferred_element_type=jnp.float32,
            precision=jax.lax.Precision.DEFAULT)
    dT = jax.lax.dot_general(
        cbm2_ref[...], z, (((1,), (1,)), ((), ())),
        preferred_element_type=jnp.float32,
        precision=jax.lax.Precision.DEFAULT) + cb2_ref[...]
    out_ref[0, 0] = jnp.argmin(dT, axis=0).astype(jnp.int32)


def kernel(x, W_enc, b_enc, codebook):
    B, C, H, W = x.shape
    w3 = W_enc.reshape(_C, _P * _P, _D)
    cbm2 = -2.0 * codebook
    cb2 = (jnp.sum(codebook * codebook, axis=-1)
           - 2.0 * (codebook @ b_enc)).reshape(_K, 1)
    out = pl.pallas_call(
        _vq_encode_kernel,
        grid=(B,),
        in_specs=[
            pl.BlockSpec((1, _C, H, W), lambda b: (b, 0, 0, 0)),
            pl.BlockSpec((_C, _P * _P, _D), lambda b: (0, 0, 0)),
            pl.BlockSpec((_K, _D), lambda b: (0, 0)),
            pl.BlockSpec((_K, 1), lambda b: (0, 0)),
        ],
        out_specs=pl.BlockSpec((1, 1, _NTOK), lambda b: (b, 0, 0)),
        out_shape=jax.ShapeDtypeStruct((B, 1, _NTOK), jnp.int32),
    )(x, w3, cbm2, cb2)
    return out.reshape(B, _NTOK // 8, 8)[:, :, ::-1].reshape(B, _NTOK).astype(jnp.uint8)


# speedup vs baseline: 6.3542x; 1.0535x over previous
import jax
import jax.numpy as jnp
from jax.experimental import pallas as pl
from jax.experimental.pallas import tpu as pltpu

_P = 16
_HP = 32
_NTOK = _HP * _HP
_K = 256
_D = 32
_C = 3


def _blockT_channel(xc, lane_masks):
    x3 = xc.reshape(64, 8, 512)
    panels = []
    for p in range(4):
        u = x3[:, :, 128 * p:128 * (p + 1)]
        u = pltpu.roll(u, 0, 2, stride=16, stride_axis=1)
        for b, mb in lane_masks:
            u = jnp.where(mb[None], pltpu.roll(u, 8 - b, 1), u)
        u = pltpu.roll(u, 8 - 1, 1)
        u = pltpu.roll(u, 16, 2)
        u = pltpu.roll(u, 0, 2, stride=16, stride_axis=1)
        panels.append(u)
    return jnp.concatenate(panels, axis=2)


def _vq_encode_kernel(x_ref, w_ref, cbm2_ref, cb2_ref, out_ref):
    gi = jax.lax.broadcasted_iota(jnp.int32, (8, 128), 1) // 16
    lane_masks = [(b, (gi // b) % 2 == 1) for b in (4, 2, 1)]
    z = jnp.zeros((8 * _HP, 4 * _D), jnp.float32)
    for c in range(_C):
        v = _blockT_channel(x_ref[0, c], lane_masks)
        v4 = v.reshape(_HP, 2, 8, 512)
        for k in range(2):
            z = z + jax.lax.dot_general(
                v4[:, k].reshape(8 * _HP, 512), w_ref[2 * c + k],
                (((1,), (0,)), ((), ())),
                preferred_element_type=jnp.float32,
                precision=jax.lax.Precision.DEFAULT)
    cbm2 = cbm2_ref[...]
    cb2 = cb2_ref[...]
    for j in range(4):
        dT = jax.lax.dot_general(
            cbm2, z[:, _D * j:_D * (j + 1)], (((1,), (1,)), ((), ())),
            preferred_element_type=jnp.float32,
            precision=jax.lax.Precision.DEFAULT) + cb2
        out_ref[0, j] = jnp.argmin(dT, axis=0).astype(jnp.int32)


def kernel(x, W_enc, b_enc, codebook):
    B, C, H, W = x.shape
    w5 = W_enc.reshape(_C, 2, 8, _P, _D)
    eye4 = jnp.eye(4, dtype=jnp.float32)
    wbd = jnp.stack([
        jnp.kron(eye4, w5[c, k].reshape(8 * _P, _D))
        for c in range(_C) for k in range(2)
    ])
    cbm2 = -2.0 * codebook
    cb2 = (jnp.sum(codebook * codebook, axis=-1)
           - 2.0 * (codebook @ b_enc)).reshape(_K, 1)
    out = pl.pallas_call(
        _vq_encode_kernel,
        grid=(B,),
        in_specs=[
            pl.BlockSpec((1, _C, H, W), lambda b: (b, 0, 0, 0)),
            pl.BlockSpec((2 * _C, 512, 128), lambda b: (0, 0, 0)),
            pl.BlockSpec((_K, _D), lambda b: (0, 0)),
            pl.BlockSpec((_K, 1), lambda b: (0, 0)),
        ],
        out_specs=pl.BlockSpec((1, 4, 8 * _HP), lambda b: (b, 0, 0)),
        out_shape=jax.ShapeDtypeStruct((B, 4, 8 * _HP), jnp.int32),
    )(x, wbd, cbm2, cb2)
    out = out.reshape(B, 4, _HP, 8)[:, :, :, ::-1]
    return out.transpose(0, 2, 1, 3).reshape(B, _NTOK).astype(jnp.uint8)
